# Initial kernel scaffold; baseline (speedup 1.0000x reference)
import jax
import jax.numpy as jnp
from jax.experimental import pallas as pl


def kernel(x, src1, dst1, d1, src2, dst2, d2, src3, dst3, d3, t1_W, t1_b, g1_1W, g1_1b, g1_2W, g1_2b, g1_3W, g1_3b, hw1_1, hw1_2, hw1_3, g2_1W, g2_1b, g2_2W, g2_2b, g2_3W, g2_3b, hw2_1, hw2_2, hw2_3, t2_W, t2_b, t3_W, t3_b):
    raise NotImplementedError("write your pallas kernel here")


# speedup vs baseline: 18.6997x; 18.6997x over previous
import functools

import jax
import jax.numpy as jnp
from jax import lax
from jax.experimental import pallas as pl
from jax.experimental.pallas import tpu as pltpu
from jax.experimental.pallas import tpu_sc as plsc

N = 100000
NFEAT = 128
NHID = 16
NCLASS = 16
EPS = 0.3
E = 1600000

NCORE = 2
NSUB = 16
NW = NCORE * NSUB
SB = 128

EP = 1605632
NB = EP // (SB * NW)
ER = EP // SB

N_PAD = 100352
RPT = N_PAD // NSUB
ZR = 784

R = 784
NBLK = N_PAD // R


def _tc_pre_body(x_ref, dd_ref, t1w_ref, t1b_ref, g1s_ref, g1d_ref, gb1_ref,
                 mid1_ref, hs0_ref, hs1_ref, hs2_ref, ps_ref, pd_ref):
    x = x_ref[...]
    x = jnp.where(jnp.isnan(x), jnp.zeros_like(x), x)
    m = jnp.dot(x, t1w_ref[...].T, preferred_element_type=jnp.float32)
    m = m + t1b_ref[...]
    mid1_ref[...] = m
    r = jax.nn.relu(m)
    ps_ref[...] = jnp.dot(r, g1s_ref[...], preferred_element_type=jnp.float32) + gb1_ref[...]
    pd_ref[...] = jnp.dot(r, g1d_ref[...], preferred_element_type=jnp.float32)
    dd = dd_ref[...]
    hs0_ref[...] = r * dd[:, 0:1]
    hs1_ref[...] = r * dd[:, 1:2]
    hs2_ref[...] = r * dd[:, 2:3]


def _tc_pre(xp, dd, t1w, t1b, g1s, g1d, gb1):
    f32 = jnp.float32
    outs = pl.pallas_call(
        _tc_pre_body,
        grid=(NBLK,),
        in_specs=[
            pl.BlockSpec((R, NFEAT), lambda i: (i, 0)),
            pl.BlockSpec((R, 3), lambda i: (i, 0)),
            pl.BlockSpec((NHID, NFEAT), lambda i: (0, 0)),
            pl.BlockSpec((1, NHID), lambda i: (0, 0)),
            pl.BlockSpec((NHID, 3), lambda i: (0, 0)),
            pl.BlockSpec((NHID, 3), lambda i: (0, 0)),
            pl.BlockSpec((1, 3), lambda i: (0, 0)),
        ],
        out_specs=[
            pl.BlockSpec((R, NHID), lambda i: (i, 0)),
            pl.BlockSpec((R, NHID), lambda i: (i, 0)),
            pl.BlockSpec((R, NHID), lambda i: (i, 0)),
            pl.BlockSpec((R, NHID), lambda i: (i, 0)),
            pl.BlockSpec((R, 3), lambda i: (i, 0)),
            pl.BlockSpec((R, 3), lambda i: (i, 0)),
        ],
        out_shape=[
            jax.ShapeDtypeStruct((N_PAD, NHID), f32),
            jax.ShapeDtypeStruct((N_PAD, NHID), f32),
            jax.ShapeDtypeStruct((N_PAD, NHID), f32),
            jax.ShapeDtypeStruct((N_PAD, NHID), f32),
            jax.ShapeDtypeStruct((N_PAD, 3), f32),
            jax.ShapeDtypeStruct((N_PAD, 3), f32),
        ],
    )(xp, dd, t1w, t1b, g1s, g1d, gb1)
    return outs


def _tc_mid_body(mid1_ref, agg_ref, dd_ref, hw1_ref, g2s_ref, g2d_ref, gb2_ref,
                 raw2_ref, hs_ref, ps_ref, pd_ref):
    r1 = jax.nn.relu(mid1_ref[...])
    dd = dd_ref[...]
    hs = [None, None, None]
    for g in range(3):
        agg = (agg_ref[g, 0] + agg_ref[g, 1]) * dd[:, g:g + 1]
        pre = EPS * r1 + agg
        hg = jax.nn.relu(jnp.dot(pre, hw1_ref[g], preferred_element_type=jnp.float32))
        hs[g] = hg
        raw2_ref[:, g * NHID:(g + 1) * NHID] = hg
    h48 = jnp.concatenate(hs, axis=1)
    ps_ref[...] = jnp.dot(h48, g2s_ref[...], preferred_element_type=jnp.float32) + gb2_ref[...]
    pd_ref[...] = jnp.dot(h48, g2d_ref[...], preferred_element_type=jnp.float32)
    for g in range(3):
        for c in range(3):
            hs_ref[3 * g + c] = hs[c] * dd[:, g:g + 1]


def _tc_mid(mid1, agg1, dd, hw1, g2s, g2d, gb2):
    f32 = jnp.float32
    return pl.pallas_call(
        _tc_mid_body,
        grid=(NBLK,),
        in_specs=[
            pl.BlockSpec((R, NHID), lambda i: (i, 0)),
            pl.BlockSpec((3, 2, R, NHID), lambda i: (0, 0, i, 0)),
            pl.BlockSpec((R, 3), lambda i: (i, 0)),
            pl.BlockSpec((3, NHID, NHID), lambda i: (0, 0, 0)),
            pl.BlockSpec((3 * NHID, 3), lambda i: (0, 0)),
            pl.BlockSpec((3 * NHID, 3), lambda i: (0, 0)),
            pl.BlockSpec((1, 3), lambda i: (0, 0)),
        ],
        out_specs=[
            pl.BlockSpec((R, 3 * NHID), lambda i: (i, 0)),
            pl.BlockSpec((9, R, NHID), lambda i: (0, i, 0)),
            pl.BlockSpec((R, 3), lambda i: (i, 0)),
            pl.BlockSpec((R, 3), lambda i: (i, 0)),
        ],
        out_shape=[
            jax.ShapeDtypeStruct((N_PAD, 3 * NHID), f32),
            jax.ShapeDtypeStruct((9, N_PAD, NHID), f32),
            jax.ShapeDtypeStruct((N_PAD, 3), f32),
            jax.ShapeDtypeStruct((N_PAD, 3), f32),
        ],
    )(mid1, agg1, dd, hw1, g2s, g2d, gb2)


def _tc_post_body(x_ref, mid1_ref, raw2_ref, dd_ref, agg_ref, hw2_ref,
                  t2w_ref, t2b_ref, t3w_ref, t3b_ref,
                  scores_ref, mid2_ref):
    x = x_ref[...]
    x = jnp.where(jnp.isnan(x), jnp.zeros_like(x), x)
    r1 = jax.nn.relu(mid1_ref[...])
    raw2 = raw2_ref[...]
    dd = dd_ref[...]
    t2w = t2w_ref[...]
    acc = jnp.zeros((x.shape[0], NHID), jnp.float32)
    for g in range(3):
        parts = []
        for c in range(3):
            p = 3 * g + c
            parts.append(agg_ref[p, 0] + agg_ref[p, 1])
        agg48 = jnp.concatenate(parts, axis=1) * dd[:, g:g + 1]
        h2g = jax.nn.relu(jnp.dot(EPS * raw2 + agg48, hw2_ref[g],
                                  preferred_element_type=jnp.float32))
        acc = acc + jnp.dot(h2g, t2w[:, g * NHID:(g + 1) * NHID].T,
                            preferred_element_type=jnp.float32)
    acc = acc + jnp.dot(r1, t2w[:, 48:64].T, preferred_element_type=jnp.float32)
    acc = acc + jnp.dot(raw2, t2w[:, 64:112].T, preferred_element_type=jnp.float32)
    acc = acc + jnp.dot(x, t2w[:, 112:240].T, preferred_element_type=jnp.float32)
    mid2 = acc + t2b_ref[...]
    mid2_ref[...] = mid2
    scores_ref[...] = jnp.dot(jax.nn.relu(mid2), t3w_ref[...].T,
                              preferred_element_type=jnp.float32) + t3b_ref[...]


def _tc_post(xp, mid1, raw2, dd, agg2, hw2, t2w, t2b, t3w, t3b):
    f32 = jnp.float32
    return pl.pallas_call(
        _tc_post_body,
        grid=(NBLK,),
        in_specs=[
            pl.BlockSpec((R, NFEAT), lambda i: (i, 0)),
            pl.BlockSpec((R, NHID), lambda i: (i, 0)),
            pl.BlockSpec((R, 3 * NHID), lambda i: (i, 0)),
            pl.BlockSpec((R, 3), lambda i: (i, 0)),
            pl.BlockSpec((9, 2, R, NHID), lambda i: (0, 0, i, 0)),
            pl.BlockSpec((3, 3 * NHID, NHID), lambda i: (0, 0, 0)),
            pl.BlockSpec((NHID, 240), lambda i: (0, 0)),
            pl.BlockSpec((1, NHID), lambda i: (0, 0)),
            pl.BlockSpec((NCLASS, NHID), lambda i: (0, 0)),
            pl.BlockSpec((1, NCLASS), lambda i: (0, 0)),
        ],
        out_specs=[
            pl.BlockSpec((R, NCLASS), lambda i: (i, 0)),
            pl.BlockSpec((R, NHID), lambda i: (i, 0)),
        ],
        out_shape=[
            jax.ShapeDtypeStruct((N_PAD, NCLASS), f32),
            jax.ShapeDtypeStruct((N_PAD, NHID), f32),
        ],
    )(xp, mid1, raw2, dd, agg2, hw2, t2w, t2b, t3w, t3b)


def _sc_body(P, gmap, *refs):
    srcs, dsts = refs[0], refs[1]
    psg = refs[2:5]
    pdg = refs[5:8]
    hsp = refs[8:8 + P]
    out = refs[8 + P]
    (src_i, dst_i, rows_v, ps_v, pd_v, ztmp, acc, gsem, isem) = refs[9 + P:]

    cid = lax.axis_index("c")
    sid = lax.axis_index("s")
    wid = cid * NSUB + sid
    row0 = wid * NB

    def _zloop(i, _):
        ztmp[i, :] = jnp.zeros((NHID,), jnp.float32)
        return 0
    lax.fori_loop(0, ZR, _zloop, 0)

    for p in range(P):
        g = gmap[p]
        hs = hsp[p]
        pst = psg[g]
        pdt = pdg[g]

        def _acczero(z, _):
            pltpu.sync_copy(ztmp, acc.at[pl.ds(sid * RPT + z * ZR, ZR)])
            return 0
        lax.fori_loop(0, RPT // ZR, _acczero, 0)
        plsc.subcore_barrier()

        def fire_idx(j, bi):
            pltpu.async_copy(srcs.at[g, row0 + j], src_i.at[bi], isem.at[bi])
            pltpu.async_copy(dsts.at[g, row0 + j], dst_i.at[bi], isem.at[bi])

        def wait_idx(j, bi):
            pltpu.make_async_copy(srcs.at[g, row0 + j], src_i.at[bi], isem.at[bi]).wait()
            pltpu.make_async_copy(dsts.at[g, row0 + j], dst_i.at[bi], isem.at[bi]).wait()

        def fire(bi, b):
            pltpu.async_copy(hs.at[src_i.at[bi]], rows_v.at[b], gsem.at[b])
            pltpu.async_copy(pst.at[src_i.at[bi]], ps_v.at[b], gsem.at[b])
            pltpu.async_copy(pdt.at[dst_i.at[bi]], pd_v.at[b], gsem.at[b])

        def wait_gathers(bi, b):
            pltpu.make_async_copy(hs.at[src_i.at[bi]], rows_v.at[b], gsem.at[b]).wait()
            pltpu.make_async_copy(pst.at[src_i.at[bi]], ps_v.at[b], gsem.at[b]).wait()
            pltpu.make_async_copy(pdt.at[dst_i.at[bi]], pd_v.at[b], gsem.at[b]).wait()

        def process(bi, b):
            wait_gathers(bi, b)

            def _rloop(v, _):
                sl = pl.ds(v * 16, 16)
                z = pd_v[b, sl] + ps_v[b, sl]
                ev = 1.0 - 2.0 / (jnp.exp(z + z) + 1.0)
                base = v * 16
                for k in range(16):
                    rows_v[b, base + k, :] = rows_v[b, base + k, :] * ev[k]
                return 0
            lax.fori_loop(0, SB // 16, _rloop, 0)
            pltpu.sync_copy(rows_v.at[b], acc.at[dst_i.at[bi]], add=True)

        fire_idx(0, 0)
        fire_idx(1, 1)
        wait_idx(0, 0)
        fire(0, 0)

        def outer(k, _):
            for u in range(4):
                j = 4 * k + u
                b = u % 2

                @pl.when(j + 2 < NB)
                def _():
                    fire_idx(j + 2, (u + 2) % 4)

                @pl.when(j + 1 < NB)
                def _():
                    wait_idx(j + 1, (u + 1) % 4)
                    fire((u + 1) % 4, 1 - b)

                process(u, b)
            return 0
        lax.fori_loop(0, NB // 4, outer, 0)

        plsc.subcore_barrier()
        pltpu.sync_copy(acc.at[pl.ds(sid * RPT, RPT)],
                        out.at[p, cid, pl.ds(sid * RPT, RPT)])
        plsc.subcore_barrier()


def _sc_passes(P, gmap, srcs, dsts, ps_list, pd_list, hs_list):
    body = functools.partial(_sc_body, P, gmap)
    kern = pl.kernel(
        body,
        out_type=jax.ShapeDtypeStruct((P, 2, N_PAD, NHID), jnp.float32),
        mesh=plsc.VectorSubcoreMesh(core_axis_name="c", subcore_axis_name="s",
                                    num_cores=NCORE, num_subcores=NSUB),
        scratch_types=[
            pltpu.VMEM((4, SB), jnp.int32),
            pltpu.VMEM((4, SB), jnp.int32),
            pltpu.VMEM((2, SB, NHID), jnp.float32),
            pltpu.VMEM((2, SB), jnp.float32),
            pltpu.VMEM((2, SB), jnp.float32),
            pltpu.VMEM((ZR, NHID), jnp.float32),
            pltpu.VMEM_SHARED((N_PAD, NHID), jnp.float32),
            pltpu.SemaphoreType.DMA((2,)),
            pltpu.SemaphoreType.DMA((4,)),
        ],
        compiler_params=pltpu.CompilerParams(use_tc_tiling_on_sc=False),
    )
    return kern(srcs, dsts, *ps_list, *pd_list, *hs_list)


def _pad_edges(a):
    a = a.astype(jnp.int32)
    return jnp.pad(a, (0, EP - E), constant_values=N).reshape(ER, SB)


def kernel(x, src1, dst1, d1, src2, dst2, d2, src3, dst3, d3,
           t1_W, t1_b, g1_1W, g1_1b, g1_2W, g1_2b, g1_3W, g1_3b,
           hw1_1, hw1_2, hw1_3,
           g2_1W, g2_1b, g2_2W, g2_2b, g2_3W, g2_3b,
           hw2_1, hw2_2, hw2_3, t2_W, t2_b, t3_W, t3_b):
    f32 = jnp.float32
    xp = jnp.pad(x.astype(f32), ((0, N_PAD - N), (0, 0)))
    dd = jnp.pad(jnp.stack([d1, d2, d3], axis=1).astype(f32),
                 ((0, N_PAD - N), (0, 0)))
    srcs = jnp.stack([_pad_edges(src1), _pad_edges(src2), _pad_edges(src3)])
    dsts = jnp.stack([_pad_edges(dst1), _pad_edges(dst2), _pad_edges(dst3)])

    g1s = jnp.stack([g1_1W[0, NHID:], g1_2W[0, NHID:], g1_3W[0, NHID:]], axis=1)
    g1d = jnp.stack([g1_1W[0, :NHID], g1_2W[0, :NHID], g1_3W[0, :NHID]], axis=1)
    gb1 = jnp.stack([g1_1b[0], g1_2b[0], g1_3b[0]]).reshape(1, 3)
    g2s = jnp.stack([g2_1W[0, 48:], g2_2W[0, 48:], g2_3W[0, 48:]], axis=1)
    g2d = jnp.stack([g2_1W[0, :48], g2_2W[0, :48], g2_3W[0, :48]], axis=1)
    gb2 = jnp.stack([g2_1b[0], g2_2b[0], g2_3b[0]]).reshape(1, 3)

    mid1, hsa, hsb, hsc, ps1, pd1 = _tc_pre(
        xp, dd, t1_W, t1_b.reshape(1, NHID), g1s, g1d, gb1)

    agg1 = _sc_passes(
        3, (0, 1, 2), srcs, dsts,
        [ps1[:, 0], ps1[:, 1], ps1[:, 2]],
        [pd1[:, 0], pd1[:, 1], pd1[:, 2]],
        [hsa, hsb, hsc])

    hw1 = jnp.stack([hw1_1, hw1_2, hw1_3])
    raw2, hs2, ps2, pd2 = _tc_mid(mid1, agg1, dd, hw1, g2s, g2d, gb2)

    agg2 = _sc_passes(
        9, (0, 0, 0, 1, 1, 1, 2, 2, 2), srcs, dsts,
        [ps2[:, 0], ps2[:, 1], ps2[:, 2]],
        [pd2[:, 0], pd2[:, 1], pd2[:, 2]],
        [hs2[p] for p in range(9)])

    hw2 = jnp.stack([hw2_1, hw2_2, hw2_3])
    scores, mid2 = _tc_post(xp, mid1, raw2, dd, agg2, hw2,
                            t2_W, t2_b.reshape(1, NHID), t3_W,
                            t3_b.reshape(1, NCLASS))
    return (scores[:N], mid1[:N], mid2[:N])


# speedup vs baseline: 19.4205x; 1.0385x over previous
import functools

import jax
import jax.numpy as jnp
from jax import lax
from jax.experimental import pallas as pl
from jax.experimental.pallas import tpu as pltpu
from jax.experimental.pallas import tpu_sc as plsc

N = 100000
NFEAT = 128
NHID = 16
NCLASS = 16
EPS = 0.3
E = 1600000

NCORE = 2
NSUB = 16
NW = NCORE * NSUB
SB = 128

EP = 1605632
NB = EP // (SB * NW)
ER = EP // SB

N_PAD = 100352
RPT = N_PAD // NSUB
ZR = 784

R = 784
NBLK = N_PAD // R


def _tc_pre_body(x_ref, dd_ref, t1w_ref, t1b_ref, g1s_ref, g1d_ref, gb1_ref,
                 mid1_ref, hs0_ref, hs1_ref, hs2_ref, ps_ref, pd_ref):
    x = x_ref[...]
    x = jnp.where(jnp.isnan(x), jnp.zeros_like(x), x)
    m = jnp.dot(x, t1w_ref[...].T, preferred_element_type=jnp.float32)
    m = m + t1b_ref[...]
    mid1_ref[...] = m
    r = jax.nn.relu(m)
    ps_ref[...] = jnp.dot(r, g1s_ref[...], preferred_element_type=jnp.float32) + gb1_ref[...]
    pd_ref[...] = jnp.dot(r, g1d_ref[...], preferred_element_type=jnp.float32)
    dd = dd_ref[...]
    hs0_ref[...] = r * dd[:, 0:1]
    hs1_ref[...] = r * dd[:, 1:2]
    hs2_ref[...] = r * dd[:, 2:3]


def _tc_pre(xp, dd, t1w, t1b, g1s, g1d, gb1):
    f32 = jnp.float32
    outs = pl.pallas_call(
        _tc_pre_body,
        grid=(NBLK,),
        in_specs=[
            pl.BlockSpec((R, NFEAT), lambda i: (i, 0)),
            pl.BlockSpec((R, 3), lambda i: (i, 0)),
            pl.BlockSpec((NHID, NFEAT), lambda i: (0, 0)),
            pl.BlockSpec((1, NHID), lambda i: (0, 0)),
            pl.BlockSpec((NHID, 3), lambda i: (0, 0)),
            pl.BlockSpec((NHID, 3), lambda i: (0, 0)),
            pl.BlockSpec((1, 3), lambda i: (0, 0)),
        ],
        out_specs=[
            pl.BlockSpec((R, NHID), lambda i: (i, 0)),
            pl.BlockSpec((R, NHID), lambda i: (i, 0)),
            pl.BlockSpec((R, NHID), lambda i: (i, 0)),
            pl.BlockSpec((R, NHID), lambda i: (i, 0)),
            pl.BlockSpec((R, 3), lambda i: (i, 0)),
            pl.BlockSpec((R, 3), lambda i: (i, 0)),
        ],
        out_shape=[
            jax.ShapeDtypeStruct((N_PAD, NHID), f32),
            jax.ShapeDtypeStruct((N_PAD, NHID), f32),
            jax.ShapeDtypeStruct((N_PAD, NHID), f32),
            jax.ShapeDtypeStruct((N_PAD, NHID), f32),
            jax.ShapeDtypeStruct((N_PAD, 3), f32),
            jax.ShapeDtypeStruct((N_PAD, 3), f32),
        ],
    )(xp, dd, t1w, t1b, g1s, g1d, gb1)
    return outs


def _tc_mid_body(mid1_ref, agg_ref, dd_ref, hw1_ref, g2s_ref, g2d_ref, gb2_ref,
                 raw2_ref, hs_ref, ps_ref, pd_ref):
    r1 = jax.nn.relu(mid1_ref[...])
    dd = dd_ref[...]
    hs = [None, None, None]
    for g in range(3):
        agg = (agg_ref[g, 0] + agg_ref[g, 1]) * dd[:, g:g + 1]
        pre = EPS * r1 + agg
        hg = jax.nn.relu(jnp.dot(pre, hw1_ref[g], preferred_element_type=jnp.float32))
        hs[g] = hg
        raw2_ref[:, g * NHID:(g + 1) * NHID] = hg
    h48 = jnp.concatenate(hs, axis=1)
    ps_ref[...] = jnp.dot(h48, g2s_ref[...], preferred_element_type=jnp.float32) + gb2_ref[...]
    pd_ref[...] = jnp.dot(h48, g2d_ref[...], preferred_element_type=jnp.float32)
    for g in range(3):
        for c in range(3):
            hs_ref[3 * g + c] = hs[c] * dd[:, g:g + 1]


def _tc_mid(mid1, agg1, dd, hw1, g2s, g2d, gb2):
    f32 = jnp.float32
    return pl.pallas_call(
        _tc_mid_body,
        grid=(NBLK,),
        in_specs=[
            pl.BlockSpec((R, NHID), lambda i: (i, 0)),
            pl.BlockSpec((3, 2, R, NHID), lambda i: (0, 0, i, 0)),
            pl.BlockSpec((R, 3), lambda i: (i, 0)),
            pl.BlockSpec((3, NHID, NHID), lambda i: (0, 0, 0)),
            pl.BlockSpec((3 * NHID, 3), lambda i: (0, 0)),
            pl.BlockSpec((3 * NHID, 3), lambda i: (0, 0)),
            pl.BlockSpec((1, 3), lambda i: (0, 0)),
        ],
        out_specs=[
            pl.BlockSpec((R, 3 * NHID), lambda i: (i, 0)),
            pl.BlockSpec((9, R, NHID), lambda i: (0, i, 0)),
            pl.BlockSpec((R, 3), lambda i: (i, 0)),
            pl.BlockSpec((R, 3), lambda i: (i, 0)),
        ],
        out_shape=[
            jax.ShapeDtypeStruct((N_PAD, 3 * NHID), f32),
            jax.ShapeDtypeStruct((9, N_PAD, NHID), f32),
            jax.ShapeDtypeStruct((N_PAD, 3), f32),
            jax.ShapeDtypeStruct((N_PAD, 3), f32),
        ],
    )(mid1, agg1, dd, hw1, g2s, g2d, gb2)


def _tc_post_body(x_ref, mid1_ref, raw2_ref, dd_ref, agg_ref, hw2_ref,
                  t2w_ref, t2b_ref, t3w_ref, t3b_ref,
                  scores_ref, mid2_ref):
    x = x_ref[...]
    x = jnp.where(jnp.isnan(x), jnp.zeros_like(x), x)
    r1 = jax.nn.relu(mid1_ref[...])
    raw2 = raw2_ref[...]
    dd = dd_ref[...]
    t2w = t2w_ref[...]
    acc = jnp.zeros((x.shape[0], NHID), jnp.float32)
    for g in range(3):
        parts = []
        for c in range(3):
            p = 3 * g + c
            parts.append(agg_ref[p, 0] + agg_ref[p, 1])
        agg48 = jnp.concatenate(parts, axis=1) * dd[:, g:g + 1]
        h2g = jax.nn.relu(jnp.dot(EPS * raw2 + agg48, hw2_ref[g],
                                  preferred_element_type=jnp.float32))
        acc = acc + jnp.dot(h2g, t2w[:, g * NHID:(g + 1) * NHID].T,
                            preferred_element_type=jnp.float32)
    acc = acc + jnp.dot(r1, t2w[:, 48:64].T, preferred_element_type=jnp.float32)
    acc = acc + jnp.dot(raw2, t2w[:, 64:112].T, preferred_element_type=jnp.float32)
    acc = acc + jnp.dot(x, t2w[:, 112:240].T, preferred_element_type=jnp.float32)
    mid2 = acc + t2b_ref[...]
    mid2_ref[...] = mid2
    scores_ref[...] = jnp.dot(jax.nn.relu(mid2), t3w_ref[...].T,
                              preferred_element_type=jnp.float32) + t3b_ref[...]


def _tc_post(xp, mid1, raw2, dd, agg2, hw2, t2w, t2b, t3w, t3b):
    f32 = jnp.float32
    return pl.pallas_call(
        _tc_post_body,
        grid=(NBLK,),
        in_specs=[
            pl.BlockSpec((R, NFEAT), lambda i: (i, 0)),
            pl.BlockSpec((R, NHID), lambda i: (i, 0)),
            pl.BlockSpec((R, 3 * NHID), lambda i: (i, 0)),
            pl.BlockSpec((R, 3), lambda i: (i, 0)),
            pl.BlockSpec((9, 2, R, NHID), lambda i: (0, 0, i, 0)),
            pl.BlockSpec((3, 3 * NHID, NHID), lambda i: (0, 0, 0)),
            pl.BlockSpec((NHID, 240), lambda i: (0, 0)),
            pl.BlockSpec((1, NHID), lambda i: (0, 0)),
            pl.BlockSpec((NCLASS, NHID), lambda i: (0, 0)),
            pl.BlockSpec((1, NCLASS), lambda i: (0, 0)),
        ],
        out_specs=[
            pl.BlockSpec((R, NCLASS), lambda i: (i, 0)),
            pl.BlockSpec((R, NHID), lambda i: (i, 0)),
        ],
        out_shape=[
            jax.ShapeDtypeStruct((N_PAD, NCLASS), f32),
            jax.ShapeDtypeStruct((N_PAD, NHID), f32),
        ],
    )(xp, mid1, raw2, dd, agg2, hw2, t2w, t2b, t3w, t3b)


def _sc_body(P, gmap, *refs):
    srcs, dsts = refs[0], refs[1]
    psg = refs[2:5]
    pdg = refs[5:8]
    hsp = refs[8:8 + P]
    out = refs[8 + P]
    (src_i, dst_i, rows_v, ps_v, pd_v, ztmp, acc, gsem, isem, ssem) = refs[9 + P:]

    cid = lax.axis_index("c")
    sid = lax.axis_index("s")
    wid = cid * NSUB + sid
    row0 = wid * NB

    def _zloop(i, _):
        ztmp[i, :] = jnp.zeros((NHID,), jnp.float32)
        return 0
    lax.fori_loop(0, ZR, _zloop, 0)

    for p in range(P):
        g = gmap[p]
        hs = hsp[p]
        pst = psg[g]
        pdt = pdg[g]

        def _acczero(z, _):
            pltpu.sync_copy(ztmp, acc.at[pl.ds(sid * RPT + z * ZR, ZR)])
            return 0
        lax.fori_loop(0, RPT // ZR, _acczero, 0)
        plsc.subcore_barrier()

        def fire_idx(j, bi):
            pltpu.async_copy(srcs.at[g, row0 + j], src_i.at[bi], isem.at[bi])
            pltpu.async_copy(dsts.at[g, row0 + j], dst_i.at[bi], isem.at[bi])

        def wait_idx(j, bi):
            pltpu.make_async_copy(srcs.at[g, row0 + j], src_i.at[bi], isem.at[bi]).wait()
            pltpu.make_async_copy(dsts.at[g, row0 + j], dst_i.at[bi], isem.at[bi]).wait()

        def fire(b):
            pltpu.async_copy(hs.at[src_i.at[b]], rows_v.at[b], gsem.at[b])
            pltpu.async_copy(pst.at[src_i.at[b]], ps_v.at[b], gsem.at[b])
            pltpu.async_copy(pdt.at[dst_i.at[b]], pd_v.at[b], gsem.at[b])

        def wait_gathers(b):
            pltpu.make_async_copy(hs.at[src_i.at[b]], rows_v.at[b], gsem.at[b]).wait()
            pltpu.make_async_copy(pst.at[src_i.at[b]], ps_v.at[b], gsem.at[b]).wait()
            pltpu.make_async_copy(pdt.at[dst_i.at[b]], pd_v.at[b], gsem.at[b]).wait()

        def fire_scat(b):
            pltpu.async_copy(rows_v.at[b], acc.at[dst_i.at[b]], ssem.at[b], add=True)

        def wait_scat(b):
            pltpu.make_async_copy(rows_v.at[b], acc.at[dst_i.at[b]], ssem.at[b]).wait()

        def process(b):
            wait_gathers(b)

            def _rloop(v, _):
                sl = pl.ds(v * 16, 16)
                z = pd_v[b, sl] + ps_v[b, sl]
                ev = 1.0 - 2.0 / (jnp.exp(z + z) + 1.0)
                base = v * 16
                for k in range(16):
                    rows_v[b, base + k, :] = rows_v[b, base + k, :] * ev[k]
                return 0
            lax.fori_loop(0, SB // 16, _rloop, 0)
            fire_scat(b)

        fire_idx(0, 0)
        fire_idx(1, 1)
        wait_idx(0, 0)
        fire(0)

        def outer(k, _):
            for u in range(4):
                j = 4 * k + u

                @pl.when((j >= 2) & (j + 2 < NB))
                def _():
                    wait_scat((u + 2) % 4)

                @pl.when(j + 2 < NB)
                def _():
                    fire_idx(j + 2, (u + 2) % 4)

                @pl.when(j + 1 < NB)
                def _():
                    wait_idx(j + 1, (u + 1) % 4)
                    fire((u + 1) % 4)

                process(u)
            return 0
        lax.fori_loop(0, NB // 4, outer, 0)

        for s in range(4):
            wait_scat(s)

        plsc.subcore_barrier()
        pltpu.sync_copy(acc.at[pl.ds(sid * RPT, RPT)],
                        out.at[p, cid, pl.ds(sid * RPT, RPT)])
        plsc.subcore_barrier()


def _sc_passes(P, gmap, srcs, dsts, ps_list, pd_list, hs_list):
    body = functools.partial(_sc_body, P, gmap)
    kern = pl.kernel(
        body,
        out_type=jax.ShapeDtypeStruct((P, 2, N_PAD, NHID), jnp.float32),
        mesh=plsc.VectorSubcoreMesh(core_axis_name="c", subcore_axis_name="s",
                                    num_cores=NCORE, num_subcores=NSUB),
        scratch_types=[
            pltpu.VMEM((4, SB), jnp.int32),
            pltpu.VMEM((4, SB), jnp.int32),
            pltpu.VMEM((4, SB, NHID), jnp.float32),
            pltpu.VMEM((4, SB), jnp.float32),
            pltpu.VMEM((4, SB), jnp.float32),
            pltpu.VMEM((ZR, NHID), jnp.float32),
            pltpu.VMEM_SHARED((N_PAD, NHID), jnp.float32),
            pltpu.SemaphoreType.DMA((4,)),
            pltpu.SemaphoreType.DMA((4,)),
            pltpu.SemaphoreType.DMA((4,)),
        ],
        compiler_params=pltpu.CompilerParams(use_tc_tiling_on_sc=False),
    )
    return kern(srcs, dsts, *ps_list, *pd_list, *hs_list)


def _pad_edges(a):
    a = a.astype(jnp.int32)
    return jnp.pad(a, (0, EP - E), constant_values=N).reshape(ER, SB)


def kernel(x, src1, dst1, d1, src2, dst2, d2, src3, dst3, d3,
           t1_W, t1_b, g1_1W, g1_1b, g1_2W, g1_2b, g1_3W, g1_3b,
           hw1_1, hw1_2, hw1_3,
           g2_1W, g2_1b, g2_2W, g2_2b, g2_3W, g2_3b,
           hw2_1, hw2_2, hw2_3, t2_W, t2_b, t3_W, t3_b):
    f32 = jnp.float32
    xp = jnp.pad(x.astype(f32), ((0, N_PAD - N), (0, 0)))
    dd = jnp.pad(jnp.stack([d1, d2, d3], axis=1).astype(f32),
                 ((0, N_PAD - N), (0, 0)))
    srcs = jnp.stack([_pad_edges(src1), _pad_edges(src2), _pad_edges(src3)])
    dsts = jnp.stack([_pad_edges(dst1), _pad_edges(dst2), _pad_edges(dst3)])

    g1s = jnp.stack([g1_1W[0, NHID:], g1_2W[0, NHID:], g1_3W[0, NHID:]], axis=1)
    g1d = jnp.stack([g1_1W[0, :NHID], g1_2W[0, :NHID], g1_3W[0, :NHID]], axis=1)
    gb1 = jnp.stack([g1_1b[0], g1_2b[0], g1_3b[0]]).reshape(1, 3)
    g2s = jnp.stack([g2_1W[0, 48:], g2_2W[0, 48:], g2_3W[0, 48:]], axis=1)
    g2d = jnp.stack([g2_1W[0, :48], g2_2W[0, :48], g2_3W[0, :48]], axis=1)
    gb2 = jnp.stack([g2_1b[0], g2_2b[0], g2_3b[0]]).reshape(1, 3)

    mid1, hsa, hsb, hsc, ps1, pd1 = _tc_pre(
        xp, dd, t1_W, t1_b.reshape(1, NHID), g1s, g1d, gb1)

    agg1 = _sc_passes(
        3, (0, 1, 2), srcs, dsts,
        [ps1[:, 0], ps1[:, 1], ps1[:, 2]],
        [pd1[:, 0], pd1[:, 1], pd1[:, 2]],
        [hsa, hsb, hsc])

    hw1 = jnp.stack([hw1_1, hw1_2, hw1_3])
    raw2, hs2, ps2, pd2 = _tc_mid(mid1, agg1, dd, hw1, g2s, g2d, gb2)

    agg2 = _sc_passes(
        9, (0, 0, 0, 1, 1, 1, 2, 2, 2), srcs, dsts,
        [ps2[:, 0], ps2[:, 1], ps2[:, 2]],
        [pd2[:, 0], pd2[:, 1], pd2[:, 2]],
        [hs2[p] for p in range(9)])

    hw2 = jnp.stack([hw2_1, hw2_2, hw2_3])
    scores, mid2 = _tc_post(xp, mid1, raw2, dd, agg2, hw2,
                            t2_W, t2_b.reshape(1, NHID), t3_W,
                            t3_b.reshape(1, NCLASS))
    return (scores[:N], mid1[:N], mid2[:N])


# speedup vs baseline: 21.0877x; 1.0858x over previous
import functools

import jax
import jax.numpy as jnp
from jax import lax
from jax.experimental import pallas as pl
from jax.experimental.pallas import tpu as pltpu
from jax.experimental.pallas import tpu_sc as plsc

N = 100000
NFEAT = 128
NHID = 16
NCLASS = 16
EPS = 0.3
E = 1600000

NCORE = 2
NSUB = 16
NW = NCORE * NSUB
SB = 128

EROWS = E // SB
NBMIN = EROWS // NW
EXTRA = EROWS - NBMIN * NW
TRIPS = (NBMIN + 1 + 3) // 4

N_PAD = 100352
RPT = N_PAD // NSUB
ZR = 128

R = 784
NBLK = 128


def _tc_pre_body(x_ref, dd_ref, t1w_ref, t1b_ref, g1s_ref, g1d_ref, gb1_ref,
                 mid1_ref, hs0_ref, hs1_ref, hs2_ref, ps_ref, pd_ref):
    x = x_ref[...]
    x = jnp.where(jnp.isnan(x), jnp.zeros_like(x), x)
    m = jnp.dot(x, t1w_ref[...].T, preferred_element_type=jnp.float32)
    m = m + t1b_ref[...]
    mid1_ref[...] = m
    r = jax.nn.relu(m)
    ps_ref[...] = jnp.dot(r, g1s_ref[...], preferred_element_type=jnp.float32) + gb1_ref[...]
    pd_ref[...] = jnp.dot(r, g1d_ref[...], preferred_element_type=jnp.float32)
    dd = dd_ref[...]
    hs0_ref[...] = r * dd[:, 0:1]
    hs1_ref[...] = r * dd[:, 1:2]
    hs2_ref[...] = r * dd[:, 2:3]


def _tc_pre(x, dd, t1w, t1b, g1s, g1d, gb1):
    f32 = jnp.float32
    return pl.pallas_call(
        _tc_pre_body,
        grid=(NBLK,),
        in_specs=[
            pl.BlockSpec((R, NFEAT), lambda i: (i, 0)),
            pl.BlockSpec((R, 3), lambda i: (i, 0)),
            pl.BlockSpec((NHID, NFEAT), lambda i: (0, 0)),
            pl.BlockSpec((1, NHID), lambda i: (0, 0)),
            pl.BlockSpec((NHID, 3), lambda i: (0, 0)),
            pl.BlockSpec((NHID, 3), lambda i: (0, 0)),
            pl.BlockSpec((1, 3), lambda i: (0, 0)),
        ],
        out_specs=[pl.BlockSpec((R, NHID), lambda i: (i, 0))] * 4
        + [pl.BlockSpec((R, 3), lambda i: (i, 0))] * 2,
        out_shape=[jax.ShapeDtypeStruct((N, NHID), f32)] * 4
        + [jax.ShapeDtypeStruct((N, 3), f32)] * 2,
    )(x, dd, t1w, t1b, g1s, g1d, gb1)


def _tc_mid_body(mid1_ref, agg_ref, dd_ref, hw1_ref, g2s_ref, g2d_ref, gb2_ref,
                 raw2_ref, h0_ref, h1_ref, h2_ref, h3_ref, h4_ref, h5_ref,
                 h6_ref, h7_ref, h8_ref, ps_ref, pd_ref):
    hs_refs = (h0_ref, h1_ref, h2_ref, h3_ref, h4_ref, h5_ref, h6_ref,
               h7_ref, h8_ref)
    r1 = jax.nn.relu(mid1_ref[...])
    dd = dd_ref[...]
    hs = [None, None, None]
    for g in range(3):
        agg = (agg_ref[g, 0] + agg_ref[g, 1]) * dd[:, g:g + 1]
        pre = EPS * r1 + agg
        hg = jax.nn.relu(jnp.dot(pre, hw1_ref[g], preferred_element_type=jnp.float32))
        hs[g] = hg
        raw2_ref[:, g * NHID:(g + 1) * NHID] = hg
    h48 = jnp.concatenate(hs, axis=1)
    ps_ref[...] = jnp.dot(h48, g2s_ref[...], preferred_element_type=jnp.float32) + gb2_ref[...]
    pd_ref[...] = jnp.dot(h48, g2d_ref[...], preferred_element_type=jnp.float32)
    for g in range(3):
        for c in range(3):
            hs_refs[3 * g + c][...] = hs[c] * dd[:, g:g + 1]


def _tc_mid(mid1, agg1, dd, hw1, g2s, g2d, gb2):
    f32 = jnp.float32
    return pl.pallas_call(
        _tc_mid_body,
        grid=(NBLK,),
        in_specs=[
            pl.BlockSpec((R, NHID), lambda i: (i, 0)),
            pl.BlockSpec((3, 2, R, NHID), lambda i: (0, 0, i, 0)),
            pl.BlockSpec((R, 3), lambda i: (i, 0)),
            pl.BlockSpec((3, NHID, NHID), lambda i: (0, 0, 0)),
            pl.BlockSpec((3 * NHID, 3), lambda i: (0, 0)),
            pl.BlockSpec((3 * NHID, 3), lambda i: (0, 0)),
            pl.BlockSpec((1, 3), lambda i: (0, 0)),
        ],
        out_specs=[pl.BlockSpec((R, 3 * NHID), lambda i: (i, 0))]
        + [pl.BlockSpec((R, NHID), lambda i: (i, 0))] * 9
        + [pl.BlockSpec((R, 3), lambda i: (i, 0))] * 2,
        out_shape=[jax.ShapeDtypeStruct((N, 3 * NHID), f32)]
        + [jax.ShapeDtypeStruct((N, NHID), f32)] * 9
        + [jax.ShapeDtypeStruct((N, 3), f32)] * 2,
    )(mid1, agg1, dd, hw1, g2s, g2d, gb2)


def _tc_post_body(x_ref, mid1_ref, raw2_ref, dd_ref, agg_ref, hw2_ref,
                  t2w_ref, t2b_ref, t3w_ref, t3b_ref,
                  scores_ref, mid2_ref):
    x = x_ref[...]
    x = jnp.where(jnp.isnan(x), jnp.zeros_like(x), x)
    r1 = jax.nn.relu(mid1_ref[...])
    raw2 = raw2_ref[...]
    dd = dd_ref[...]
    t2w = t2w_ref[...]
    acc = jnp.zeros((x.shape[0], NHID), jnp.float32)
    for g in range(3):
        parts = []
        for c in range(3):
            p = 3 * g + c
            parts.append(agg_ref[p, 0] + agg_ref[p, 1])
        agg48 = jnp.concatenate(parts, axis=1) * dd[:, g:g + 1]
        h2g = jax.nn.relu(jnp.dot(EPS * raw2 + agg48, hw2_ref[g],
                                  preferred_element_type=jnp.float32))
        acc = acc + jnp.dot(h2g, t2w[:, g * NHID:(g + 1) * NHID].T,
                            preferred_element_type=jnp.float32)
    acc = acc + jnp.dot(r1, t2w[:, 48:64].T, preferred_element_type=jnp.float32)
    acc = acc + jnp.dot(raw2, t2w[:, 64:112].T, preferred_element_type=jnp.float32)
    acc = acc + jnp.dot(x, t2w[:, 112:240].T, preferred_element_type=jnp.float32)
    mid2 = acc + t2b_ref[...]
    mid2_ref[...] = mid2
    scores_ref[...] = jnp.dot(jax.nn.relu(mid2), t3w_ref[...].T,
                              preferred_element_type=jnp.float32) + t3b_ref[...]


def _tc_post(x, mid1, raw2, dd, agg2, hw2, t2w, t2b, t3w, t3b):
    f32 = jnp.float32
    return pl.pallas_call(
        _tc_post_body,
        grid=(NBLK,),
        in_specs=[
            pl.BlockSpec((R, NFEAT), lambda i: (i, 0)),
            pl.BlockSpec((R, NHID), lambda i: (i, 0)),
            pl.BlockSpec((R, 3 * NHID), lambda i: (i, 0)),
            pl.BlockSpec((R, 3), lambda i: (i, 0)),
            pl.BlockSpec((9, 2, R, NHID), lambda i: (0, 0, i, 0)),
            pl.BlockSpec((3, 3 * NHID, NHID), lambda i: (0, 0, 0)),
            pl.BlockSpec((NHID, 240), lambda i: (0, 0)),
            pl.BlockSpec((1, NHID), lambda i: (0, 0)),
            pl.BlockSpec((NCLASS, NHID), lambda i: (0, 0)),
            pl.BlockSpec((1, NCLASS), lambda i: (0, 0)),
        ],
        out_specs=[
            pl.BlockSpec((R, NCLASS), lambda i: (i, 0)),
            pl.BlockSpec((R, NHID), lambda i: (i, 0)),
        ],
        out_shape=[
            jax.ShapeDtypeStruct((N, NCLASS), f32),
            jax.ShapeDtypeStruct((N, NHID), f32),
        ],
    )(x, mid1, raw2, dd, agg2, hw2, t2w, t2b, t3w, t3b)


def _sc_body(P, gmap, *refs):
    srcg = refs[0:3]
    dstg = refs[3:6]
    psg = refs[6:9]
    pdg = refs[9:12]
    hsp = refs[12:12 + P]
    out = refs[12 + P]
    (src_i, dst_i, rows_v, ps_v, pd_v, ztmp, acc,
     gsem, isem, ssem) = refs[13 + P:]

    cid = lax.axis_index("c")
    sid = lax.axis_index("s")
    wid = cid * NSUB + sid
    nb = jnp.where(wid < EXTRA, NBMIN + 1, NBMIN)
    row0 = wid * NBMIN + jnp.minimum(wid, EXTRA)

    def _zloop(i, _):
        ztmp[i, :] = jnp.zeros((NHID,), jnp.float32)
        return 0
    lax.fori_loop(0, ZR, _zloop, 0)

    for p in range(P):
        g = gmap[p]
        hs = hsp[p]
        pst = psg[g]
        pdt = pdg[g]
        sg = srcg[g]
        dg = dstg[g]

        def _acczero(z, _):
            pltpu.sync_copy(ztmp, acc.at[pl.ds(sid * RPT + z * ZR, ZR)])
            return 0
        lax.fori_loop(0, RPT // ZR, _acczero, 0)
        plsc.subcore_barrier()

        def fire_idx(j, bi):
            base = (row0 + j) * SB
            pltpu.async_copy(sg.at[pl.ds(base, SB)], src_i.at[bi], isem.at[bi])
            pltpu.async_copy(dg.at[pl.ds(base, SB)], dst_i.at[bi], isem.at[bi])

        def wait_idx(j, bi):
            base = (row0 + j) * SB
            pltpu.make_async_copy(sg.at[pl.ds(base, SB)], src_i.at[bi], isem.at[bi]).wait()
            pltpu.make_async_copy(dg.at[pl.ds(base, SB)], dst_i.at[bi], isem.at[bi]).wait()

        def fire(b):
            pltpu.async_copy(hs.at[src_i.at[b]], rows_v.at[b], gsem.at[b])
            pltpu.async_copy(pst.at[src_i.at[b]], ps_v.at[b], gsem.at[b])
            pltpu.async_copy(pdt.at[dst_i.at[b]], pd_v.at[b], gsem.at[b])

        def wait_gathers(b):
            pltpu.make_async_copy(hs.at[src_i.at[b]], rows_v.at[b], gsem.at[b]).wait()
            pltpu.make_async_copy(pst.at[src_i.at[b]], ps_v.at[b], gsem.at[b]).wait()
            pltpu.make_async_copy(pdt.at[dst_i.at[b]], pd_v.at[b], gsem.at[b]).wait()

        def fire_scat(b):
            pltpu.async_copy(rows_v.at[b], acc.at[dst_i.at[b]], ssem.at[b], add=True)

        def wait_scat(b):
            pltpu.make_async_copy(rows_v.at[b], acc.at[dst_i.at[b]], ssem.at[b]).wait()

        def process(b):
            wait_gathers(b)

            def _rloop(v, _):
                sl = pl.ds(v * 16, 16)
                z = pd_v[b, sl] + ps_v[b, sl]
                ev = 1.0 - 2.0 / (jnp.exp(z + z) + 1.0)
                base = v * 16
                for k in range(16):
                    rows_v[b, base + k, :] = rows_v[b, base + k, :] * ev[k]
                return 0
            lax.fori_loop(0, SB // 16, _rloop, 0)
            fire_scat(b)

        fire_idx(0, 0)
        fire_idx(1, 1)
        wait_idx(0, 0)
        fire(0)

        def outer(k, _):
            for u in range(4):
                j = 4 * k + u

                @pl.when((j >= 2) & (j + 2 < nb))
                def _():
                    wait_scat((u + 2) % 4)

                @pl.when(j + 2 < nb)
                def _():
                    fire_idx(j + 2, (u + 2) % 4)

                @pl.when(j + 1 < nb)
                def _():
                    wait_idx(j + 1, (u + 1) % 4)
                    fire((u + 1) % 4)

                @pl.when(j < nb)
                def _():
                    process(u)
            return 0
        lax.fori_loop(0, TRIPS, outer, 0)

        for s in range(4):
            wait_scat(s)

        plsc.subcore_barrier()
        pltpu.sync_copy(acc.at[pl.ds(sid * RPT, RPT)],
                        out.at[p, cid, pl.ds(sid * RPT, RPT)])
        plsc.subcore_barrier()


def _sc_passes(P, gmap, srcs, dsts, ps_list, pd_list, hs_list):
    body = functools.partial(_sc_body, P, gmap)
    kern = pl.kernel(
        body,
        out_type=jax.ShapeDtypeStruct((P, 2, N_PAD, NHID), jnp.float32),
        mesh=plsc.VectorSubcoreMesh(core_axis_name="c", subcore_axis_name="s",
                                    num_cores=NCORE, num_subcores=NSUB),
        scratch_types=[
            pltpu.VMEM((4, SB), jnp.int32),
            pltpu.VMEM((4, SB), jnp.int32),
            pltpu.VMEM((4, SB, NHID), jnp.float32),
            pltpu.VMEM((4, SB), jnp.float32),
            pltpu.VMEM((4, SB), jnp.float32),
            pltpu.VMEM((ZR, NHID), jnp.float32),
            pltpu.VMEM_SHARED((N_PAD, NHID), jnp.float32),
            pltpu.SemaphoreType.DMA((4,)),
            pltpu.SemaphoreType.DMA((4,)),
            pltpu.SemaphoreType.DMA((4,)),
        ],
        compiler_params=pltpu.CompilerParams(use_tc_tiling_on_sc=False),
    )
    return kern(*srcs, *dsts, *ps_list, *pd_list, *hs_list)


def kernel(x, src1, dst1, d1, src2, dst2, d2, src3, dst3, d3,
           t1_W, t1_b, g1_1W, g1_1b, g1_2W, g1_2b, g1_3W, g1_3b,
           hw1_1, hw1_2, hw1_3,
           g2_1W, g2_1b, g2_2W, g2_2b, g2_3W, g2_3b,
           hw2_1, hw2_2, hw2_3, t2_W, t2_b, t3_W, t3_b):
    f32 = jnp.float32
    x = x.astype(f32)
    dd = jnp.stack([d1, d2, d3], axis=1).astype(f32)
    srcs = [src1.astype(jnp.int32), src2.astype(jnp.int32), src3.astype(jnp.int32)]
    dsts = [dst1.astype(jnp.int32), dst2.astype(jnp.int32), dst3.astype(jnp.int32)]

    g1s = jnp.stack([g1_1W[0, NHID:], g1_2W[0, NHID:], g1_3W[0, NHID:]], axis=1)
    g1d = jnp.stack([g1_1W[0, :NHID], g1_2W[0, :NHID], g1_3W[0, :NHID]], axis=1)
    gb1 = jnp.stack([g1_1b[0], g1_2b[0], g1_3b[0]]).reshape(1, 3)
    g2s = jnp.stack([g2_1W[0, 48:], g2_2W[0, 48:], g2_3W[0, 48:]], axis=1)
    g2d = jnp.stack([g2_1W[0, :48], g2_2W[0, :48], g2_3W[0, :48]], axis=1)
    gb2 = jnp.stack([g2_1b[0], g2_2b[0], g2_3b[0]]).reshape(1, 3)

    mid1, hsa, hsb, hsc, ps1, pd1 = _tc_pre(
        x, dd, t1_W, t1_b.reshape(1, NHID), g1s, g1d, gb1)

    agg1 = _sc_passes(
        3, (0, 1, 2), srcs, dsts,
        [ps1[:, 0], ps1[:, 1], ps1[:, 2]],
        [pd1[:, 0], pd1[:, 1], pd1[:, 2]],
        [hsa, hsb, hsc])

    hw1 = jnp.stack([hw1_1, hw1_2, hw1_3])
    outs = _tc_mid(mid1, agg1, dd, hw1, g2s, g2d, gb2)
    raw2 = outs[0]
    hs2 = outs[1:10]
    ps2, pd2 = outs[10], outs[11]

    agg2 = _sc_passes(
        9, (0, 0, 0, 1, 1, 1, 2, 2, 2), srcs, dsts,
        [ps2[:, 0], ps2[:, 1], ps2[:, 2]],
        [pd2[:, 0], pd2[:, 1], pd2[:, 2]],
        list(hs2))

    hw2 = jnp.stack([hw2_1, hw2_2, hw2_3])
    scores, mid2 = _tc_post(x, mid1, raw2, dd, agg2, hw2,
                            t2_W, t2_b.reshape(1, NHID), t3_W,
                            t3_b.reshape(1, NCLASS))
    return (scores, mid1, mid2)


# speedup vs baseline: 23.4121x; 1.1102x over previous
import functools

import jax
import jax.numpy as jnp
from jax import lax
from jax.experimental import pallas as pl
from jax.experimental.pallas import tpu as pltpu
from jax.experimental.pallas import tpu_sc as plsc

N = 100000
NFEAT = 128
NHID = 16
NCLASS = 16
EPS = 0.3
E = 1600000

NCORE = 2
NSUB = 16
NW = NCORE * NSUB
SB = 128

EROWS = E // SB
NBMIN = EROWS // NW
EXTRA = EROWS - NBMIN * NW
TRIPS = (NBMIN + 1 + 3) // 4

N_PAD = 100352
RPT = N_PAD // NSUB
ZR = 128

R = 1024
R8 = R // 8
NBLK = 98


def _tc_pre_body(x_ref, dd_ref, t1w_ref, t1b_ref, g1s_ref, g1d_ref, gb1_ref,
                 mid1_ref, hs0_ref, hs1_ref, hs2_ref, ps_ref, pd_ref):
    x = x_ref[...]
    x = jnp.where(jnp.isnan(x), jnp.zeros_like(x), x)
    m = jnp.dot(x, t1w_ref[...].T, preferred_element_type=jnp.float32)
    m = m + t1b_ref[...]
    mid1_ref[...] = m
    r = jax.nn.relu(m)
    ps_ref[...] = jnp.dot(r, g1s_ref[...], preferred_element_type=jnp.float32) + gb1_ref[...]
    pd_ref[...] = jnp.dot(r, g1d_ref[...], preferred_element_type=jnp.float32)
    dd = dd_ref[...]
    hs0_ref[...] = r * dd[:, 0:1]
    hs1_ref[...] = r * dd[:, 1:2]
    hs2_ref[...] = r * dd[:, 2:3]


def _tc_pre(x, dd, t1w, t1b, g1s, g1d, gb1):
    f32 = jnp.float32
    return pl.pallas_call(
        _tc_pre_body,
        grid=(NBLK,),
        in_specs=[
            pl.BlockSpec((R, NFEAT), lambda i: (i, 0)),
            pl.BlockSpec((R, 3), lambda i: (i, 0)),
            pl.BlockSpec((NHID, NFEAT), lambda i: (0, 0)),
            pl.BlockSpec((1, NHID), lambda i: (0, 0)),
            pl.BlockSpec((NHID, 3), lambda i: (0, 0)),
            pl.BlockSpec((NHID, 3), lambda i: (0, 0)),
            pl.BlockSpec((1, 3), lambda i: (0, 0)),
        ],
        out_specs=[pl.BlockSpec((R, NHID), lambda i: (i, 0))] * 4
        + [pl.BlockSpec((R, 3), lambda i: (i, 0))] * 2,
        out_shape=[jax.ShapeDtypeStruct((N, NHID), f32)] * 4
        + [jax.ShapeDtypeStruct((N, 3), f32)] * 2,
    )(x, dd, t1w, t1b, g1s, g1d, gb1)


def _tc_mid_body(mid1_ref, agg_ref, dd_ref, hw1_ref, g2s_ref, g2d_ref, gb2_ref,
                 raw2_ref, h0_ref, h1_ref, h2_ref, h3_ref, h4_ref, h5_ref,
                 h6_ref, h7_ref, h8_ref, ps_ref, pd_ref):
    hs_refs = (h0_ref, h1_ref, h2_ref, h3_ref, h4_ref, h5_ref, h6_ref,
               h7_ref, h8_ref)
    r1 = jax.nn.relu(mid1_ref[...])
    dd = dd_ref[...]
    hs = [None, None, None]
    for g in range(3):
        agg = (agg_ref[g, 0] + agg_ref[g, 1]) * dd[:, g:g + 1]
        pre = EPS * r1 + agg
        hg = jax.nn.relu(jnp.dot(pre, hw1_ref[g], preferred_element_type=jnp.float32))
        hs[g] = hg
        raw2_ref[:, g * NHID:(g + 1) * NHID] = hg
    h48 = jnp.concatenate(hs, axis=1)
    ps_ref[...] = jnp.dot(h48, g2s_ref[...], preferred_element_type=jnp.float32) + gb2_ref[...]
    pd_ref[...] = jnp.dot(h48, g2d_ref[...], preferred_element_type=jnp.float32)
    for g in range(3):
        for c in range(3):
            hs_refs[3 * g + c][...] = hs[c] * dd[:, g:g + 1]


def _tc_mid(mid1, agg1, dd, hw1, g2s, g2d, gb2):
    f32 = jnp.float32
    return pl.pallas_call(
        _tc_mid_body,
        grid=(NBLK,),
        in_specs=[
            pl.BlockSpec((R, NHID), lambda i: (i, 0)),
            pl.BlockSpec((3, 2, R, NHID), lambda i: (0, 0, i, 0)),
            pl.BlockSpec((R, 3), lambda i: (i, 0)),
            pl.BlockSpec((3, NHID, NHID), lambda i: (0, 0, 0)),
            pl.BlockSpec((3 * NHID, 3), lambda i: (0, 0)),
            pl.BlockSpec((3 * NHID, 3), lambda i: (0, 0)),
            pl.BlockSpec((1, 3), lambda i: (0, 0)),
        ],
        out_specs=[pl.BlockSpec((R, 3 * NHID), lambda i: (i, 0))]
        + [pl.BlockSpec((R, NHID), lambda i: (i, 0))] * 9
        + [pl.BlockSpec((R, 3), lambda i: (i, 0))] * 2,
        out_shape=[jax.ShapeDtypeStruct((N, 3 * NHID), f32)]
        + [jax.ShapeDtypeStruct((N, NHID), f32)] * 9
        + [jax.ShapeDtypeStruct((N, 3), f32)] * 2,
    )(mid1, agg1, dd, hw1, g2s, g2d, gb2)


def _tc_post_body(x_ref, mid1_ref, raw2_ref, dd_ref, agg_ref, hw2_ref,
                  t2w_ref, t2b_ref, t3w_ref, t3b_ref,
                  scores_ref, mid2_ref):
    x = x_ref[...]
    x = jnp.where(jnp.isnan(x), jnp.zeros_like(x), x)
    r1 = jax.nn.relu(mid1_ref[...])
    raw2 = raw2_ref[...]
    dd = dd_ref[...]
    t2w = t2w_ref[...]
    acc = jnp.zeros((x.shape[0], NHID), jnp.float32)
    for g in range(3):
        parts = []
        for c in range(3):
            p = 3 * g + c
            parts.append(agg_ref[p, 0] + agg_ref[p, 1])
        agg48 = jnp.concatenate(parts, axis=1) * dd[:, g:g + 1]
        h2g = jax.nn.relu(jnp.dot(EPS * raw2 + agg48, hw2_ref[g],
                                  preferred_element_type=jnp.float32))
        acc = acc + jnp.dot(h2g, t2w[:, g * NHID:(g + 1) * NHID].T,
                            preferred_element_type=jnp.float32)
    acc = acc + jnp.dot(r1, t2w[:, 48:64].T, preferred_element_type=jnp.float32)
    acc = acc + jnp.dot(raw2, t2w[:, 64:112].T, preferred_element_type=jnp.float32)
    acc = acc + jnp.dot(x, t2w[:, 112:240].T, preferred_element_type=jnp.float32)
    mid2 = acc + t2b_ref[...]
    mid2_ref[...] = mid2
    scores_ref[...] = jnp.dot(jax.nn.relu(mid2), t3w_ref[...].T,
                              preferred_element_type=jnp.float32) + t3b_ref[...]


def _tc_post(x, mid1, raw2, dd, agg2, hw2, t2w, t2b, t3w, t3b):
    f32 = jnp.float32
    return pl.pallas_call(
        _tc_post_body,
        grid=(NBLK,),
        in_specs=[
            pl.BlockSpec((R, NFEAT), lambda i: (i, 0)),
            pl.BlockSpec((R, NHID), lambda i: (i, 0)),
            pl.BlockSpec((R, 3 * NHID), lambda i: (i, 0)),
            pl.BlockSpec((R, 3), lambda i: (i, 0)),
            pl.BlockSpec((9, 2, R, NHID), lambda i: (0, 0, i, 0)),
            pl.BlockSpec((3, 3 * NHID, NHID), lambda i: (0, 0, 0)),
            pl.BlockSpec((NHID, 240), lambda i: (0, 0)),
            pl.BlockSpec((1, NHID), lambda i: (0, 0)),
            pl.BlockSpec((NCLASS, NHID), lambda i: (0, 0)),
            pl.BlockSpec((1, NCLASS), lambda i: (0, 0)),
        ],
        out_specs=[
            pl.BlockSpec((R, NCLASS), lambda i: (i, 0)),
            pl.BlockSpec((R, NHID), lambda i: (i, 0)),
        ],
        out_shape=[
            jax.ShapeDtypeStruct((N, NCLASS), f32),
            jax.ShapeDtypeStruct((N, NHID), f32),
        ],
    )(x, mid1, raw2, dd, agg2, hw2, t2w, t2b, t3w, t3b)


def _sc_body(P, gmap, cache, *refs):
    srcg = refs[0:3]
    dstg = refs[3:6]
    psg = refs[6:9]
    pdg = refs[9:12]
    hsp = refs[12:12 + P]
    out = refs[12 + P]
    cached = any(c != 0 for c in cache)
    ec = refs[13 + P] if cached else None
    (src_i, dst_i, rows_v, ps_v, pd_v, ztmp, acc,
     gsem, isem, ssem, esem) = refs[(14 if cached else 13) + P:]

    cid = lax.axis_index("c")
    sid = lax.axis_index("s")
    wid = cid * NSUB + sid
    nb = jnp.where(wid < EXTRA, NBMIN + 1, NBMIN)
    row0 = wid * NBMIN + jnp.minimum(wid, EXTRA)

    def _zloop(i, _):
        ztmp[i, :] = jnp.zeros((NHID,), jnp.float32)
        return 0
    lax.fori_loop(0, ZR, _zloop, 0)

    for p in range(P):
        g = gmap[p]
        mode = cache[p]
        hs = hsp[p]
        pst = psg[g]
        pdt = pdg[g]
        sg = srcg[g]
        dg = dstg[g]

        def _acczero(z, _):
            pltpu.sync_copy(ztmp, acc.at[pl.ds(sid * RPT + z * ZR, ZR)])
            return 0
        lax.fori_loop(0, RPT // ZR, _acczero, 0)
        plsc.subcore_barrier()

        def fire_idx(j, bi):
            base = (row0 + j) * SB
            pltpu.async_copy(sg.at[pl.ds(base, SB)], src_i.at[bi], isem.at[bi])
            pltpu.async_copy(dg.at[pl.ds(base, SB)], dst_i.at[bi], isem.at[bi])

        def wait_idx(j, bi):
            base = (row0 + j) * SB
            pltpu.make_async_copy(sg.at[pl.ds(base, SB)], src_i.at[bi], isem.at[bi]).wait()
            pltpu.make_async_copy(dg.at[pl.ds(base, SB)], dst_i.at[bi], isem.at[bi]).wait()

        def fire(j, b):
            pltpu.async_copy(hs.at[src_i.at[b]], rows_v.at[b], gsem.at[b])
            if mode == 2:
                base = (row0 + j) * SB
                pltpu.async_copy(ec.at[g, pl.ds(base, SB)], ps_v.at[b], gsem.at[b])
            else:
                if mode == 1:
                    @pl.when(j >= 4)
                    def _():
                        bprev = (row0 + j - 4) * SB
                        pltpu.make_async_copy(
                            ps_v.at[b], ec.at[g, pl.ds(bprev, SB)], esem.at[b]).wait()
                pltpu.async_copy(pst.at[src_i.at[b]], ps_v.at[b], gsem.at[b])
                pltpu.async_copy(pdt.at[dst_i.at[b]], pd_v.at[b], gsem.at[b])

        def wait_gathers(j, b):
            pltpu.make_async_copy(hs.at[src_i.at[b]], rows_v.at[b], gsem.at[b]).wait()
            if mode == 2:
                base = (row0 + j) * SB
                pltpu.make_async_copy(ec.at[g, pl.ds(base, SB)], ps_v.at[b], gsem.at[b]).wait()
            else:
                pltpu.make_async_copy(pst.at[src_i.at[b]], ps_v.at[b], gsem.at[b]).wait()
                pltpu.make_async_copy(pdt.at[dst_i.at[b]], pd_v.at[b], gsem.at[b]).wait()

        def fire_scat(b):
            pltpu.async_copy(rows_v.at[b], acc.at[dst_i.at[b]], ssem.at[b], add=True)

        def wait_scat(b):
            pltpu.make_async_copy(rows_v.at[b], acc.at[dst_i.at[b]], ssem.at[b]).wait()

        def process(j, b):
            wait_gathers(j, b)

            def _rloop(v, _):
                sl = pl.ds(v * 16, 16)
                if mode == 2:
                    ev = ps_v[b, sl]
                else:
                    z = pd_v[b, sl] + ps_v[b, sl]
                    ev = 1.0 - 2.0 / (jnp.exp(z + z) + 1.0)
                    if mode == 1:
                        ps_v[b, sl] = ev
                base = v * 16
                for k in range(16):
                    rows_v[b, base + k, :] = rows_v[b, base + k, :] * ev[k]
                return 0
            lax.fori_loop(0, SB // 16, _rloop, 0)
            fire_scat(b)
            if mode == 1:
                base = (row0 + j) * SB
                pltpu.async_copy(ps_v.at[b], ec.at[g, pl.ds(base, SB)], esem.at[b])

        fire_idx(0, 0)
        fire_idx(1, 1)
        wait_idx(0, 0)
        fire(0, 0)

        def outer(k, _):
            for u in range(4):
                j = 4 * k + u

                @pl.when((j >= 2) & (j + 2 < nb))
                def _():
                    wait_scat((u + 2) % 4)

                @pl.when(j + 2 < nb)
                def _():
                    fire_idx(j + 2, (u + 2) % 4)

                @pl.when(j + 1 < nb)
                def _():
                    wait_idx(j + 1, (u + 1) % 4)
                    fire(j + 1, (u + 1) % 4)

                @pl.when(j < nb)
                def _():
                    process(j, u)
            return 0
        lax.fori_loop(0, TRIPS, outer, 0)

        for s in range(4):
            wait_scat(s)
        if mode == 1:
            for s in range(4):
                pltpu.make_async_copy(
                    ps_v.at[s], ec.at[g, pl.ds(row0 * SB, SB)], esem.at[s]).wait()

        plsc.subcore_barrier()
        pltpu.sync_copy(acc.at[pl.ds(sid * RPT, RPT)],
                        out.at[p, cid, pl.ds(sid * RPT, RPT)])
        plsc.subcore_barrier()


def _sc_passes(P, gmap, cache, srcs, dsts, ps_list, pd_list, hs_list):
    cached = any(c != 0 for c in cache)
    out_type = [jax.ShapeDtypeStruct((P, 2, N_PAD, NHID), jnp.float32)]
    if cached:
        out_type.append(jax.ShapeDtypeStruct((3, E), jnp.float32))
    body = functools.partial(_sc_body, P, gmap, cache)
    kern = pl.kernel(
        body,
        out_type=out_type,
        mesh=plsc.VectorSubcoreMesh(core_axis_name="c", subcore_axis_name="s",
                                    num_cores=NCORE, num_subcores=NSUB),
        scratch_types=[
            pltpu.VMEM((4, SB), jnp.int32),
            pltpu.VMEM((4, SB), jnp.int32),
            pltpu.VMEM((4, SB, NHID), jnp.float32),
            pltpu.VMEM((4, SB), jnp.float32),
            pltpu.VMEM((4, SB), jnp.float32),
            pltpu.VMEM((ZR, NHID), jnp.float32),
            pltpu.VMEM_SHARED((N_PAD, NHID), jnp.float32),
            pltpu.SemaphoreType.DMA((4,)),
            pltpu.SemaphoreType.DMA((4,)),
            pltpu.SemaphoreType.DMA((4,)),
            pltpu.SemaphoreType.DMA((4,)),
        ],
        compiler_params=pltpu.CompilerParams(use_tc_tiling_on_sc=False),
    )
    res = kern(*srcs, *dsts, *ps_list, *pd_list, *hs_list)
    return res[0]


def kernel(x, src1, dst1, d1, src2, dst2, d2, src3, dst3, d3,
           t1_W, t1_b, g1_1W, g1_1b, g1_2W, g1_2b, g1_3W, g1_3b,
           hw1_1, hw1_2, hw1_3,
           g2_1W, g2_1b, g2_2W, g2_2b, g2_3W, g2_3b,
           hw2_1, hw2_2, hw2_3, t2_W, t2_b, t3_W, t3_b):
    f32 = jnp.float32
    x = x.astype(f32)
    dd = jnp.stack([d1, d2, d3], axis=1).astype(f32)
    srcs = [src1.astype(jnp.int32), src2.astype(jnp.int32), src3.astype(jnp.int32)]
    dsts = [dst1.astype(jnp.int32), dst2.astype(jnp.int32), dst3.astype(jnp.int32)]

    g1s = jnp.stack([g1_1W[0, NHID:], g1_2W[0, NHID:], g1_3W[0, NHID:]], axis=1)
    g1d = jnp.stack([g1_1W[0, :NHID], g1_2W[0, :NHID], g1_3W[0, :NHID]], axis=1)
    gb1 = jnp.stack([g1_1b[0], g1_2b[0], g1_3b[0]]).reshape(1, 3)
    g2s = jnp.stack([g2_1W[0, 48:], g2_2W[0, 48:], g2_3W[0, 48:]], axis=1)
    g2d = jnp.stack([g2_1W[0, :48], g2_2W[0, :48], g2_3W[0, :48]], axis=1)
    gb2 = jnp.stack([g2_1b[0], g2_2b[0], g2_3b[0]]).reshape(1, 3)

    mid1, hsa, hsb, hsc, ps1, pd1 = _tc_pre(
        x, dd, t1_W, t1_b.reshape(1, NHID), g1s, g1d, gb1)

    agg1 = _sc_passes(
        3, (0, 1, 2), (0, 0, 0), srcs, dsts,
        [ps1[:, 0], ps1[:, 1], ps1[:, 2]],
        [pd1[:, 0], pd1[:, 1], pd1[:, 2]],
        [hsa, hsb, hsc])

    hw1 = jnp.stack([hw1_1, hw1_2, hw1_3])
    outs = _tc_mid(mid1, agg1, dd, hw1, g2s, g2d, gb2)
    raw2 = outs[0]
    hs2 = outs[1:10]
    ps2, pd2 = outs[10], outs[11]

    agg2 = _sc_passes(
        9, (0, 0, 0, 1, 1, 1, 2, 2, 2), (1, 2, 2, 1, 2, 2, 1, 2, 2), srcs, dsts,
        [ps2[:, 0], ps2[:, 1], ps2[:, 2]],
        [pd2[:, 0], pd2[:, 1], pd2[:, 2]],
        list(hs2))

    hw2 = jnp.stack([hw2_1, hw2_2, hw2_3])
    scores, mid2 = _tc_post(x, mid1, raw2, dd, agg2, hw2,
                            t2_W, t2_b.reshape(1, NHID), t3_W,
                            t3_b.reshape(1, NCLASS))
    return (scores, mid1, mid2)


# speedup vs baseline: 23.5304x; 1.0051x over previous
import functools

import jax
import jax.numpy as jnp
from jax import lax
from jax.experimental import pallas as pl
from jax.experimental.pallas import tpu as pltpu
from jax.experimental.pallas import tpu_sc as plsc

N = 100000
NFEAT = 128
NHID = 16
NCLASS = 16
EPS = 0.3
E = 1600000

NCORE = 2
NSUB = 16
NW = NCORE * NSUB
SB = 128

EROWS = E // SB
NBMIN = EROWS // NW
EXTRA = EROWS - NBMIN * NW
TRIPS = (NBMIN + 1 + 3) // 4

N_PAD = 100352
RPT = N_PAD // NSUB
ZR = 128

R = 1024
R8 = R // 8
NBLK = 98


def _tc_pre_body(x_ref, dd_ref, t1w_ref, t1b_ref, g1s_ref, g1d_ref, gb1_ref,
                 mid1_ref, hs0_ref, hs1_ref, hs2_ref, ps_ref, pd_ref):
    x = x_ref[...]
    x = jnp.where(jnp.isnan(x), jnp.zeros_like(x), x)
    m = jnp.dot(x, t1w_ref[...].T, preferred_element_type=jnp.float32)
    m = m + t1b_ref[...]
    mid1_ref[...] = m
    r = jax.nn.relu(m)
    ps_ref[...] = jnp.dot(r, g1s_ref[...], preferred_element_type=jnp.float32) + gb1_ref[...]
    pd_ref[...] = jnp.dot(r, g1d_ref[...], preferred_element_type=jnp.float32)
    dd = dd_ref[...]
    hs0_ref[...] = r * dd[:, 0:1]
    hs1_ref[...] = r * dd[:, 1:2]
    hs2_ref[...] = r * dd[:, 2:3]


def _tc_pre(x, dd, t1w, t1b, g1s, g1d, gb1):
    f32 = jnp.float32
    return pl.pallas_call(
        _tc_pre_body,
        grid=(NBLK,),
        in_specs=[
            pl.BlockSpec((R, NFEAT), lambda i: (i, 0)),
            pl.BlockSpec((R, 3), lambda i: (i, 0)),
            pl.BlockSpec((NHID, NFEAT), lambda i: (0, 0)),
            pl.BlockSpec((1, NHID), lambda i: (0, 0)),
            pl.BlockSpec((NHID, 3), lambda i: (0, 0)),
            pl.BlockSpec((NHID, 3), lambda i: (0, 0)),
            pl.BlockSpec((1, 3), lambda i: (0, 0)),
        ],
        out_specs=[pl.BlockSpec((R, NHID), lambda i: (i, 0))] * 4
        + [pl.BlockSpec((R, 3), lambda i: (i, 0))] * 2,
        out_shape=[jax.ShapeDtypeStruct((N, NHID), f32)] * 4
        + [jax.ShapeDtypeStruct((N, 3), f32)] * 2,
    )(x, dd, t1w, t1b, g1s, g1d, gb1)


def _tc_mid_body(mid1_ref, agg_ref, dd_ref, hw1_ref, g2s_ref, g2d_ref, gb2_ref,
                 raw2_ref, h0_ref, h1_ref, h2_ref, h3_ref, h4_ref, h5_ref,
                 h6_ref, h7_ref, h8_ref, ps_ref, pd_ref):
    hs_refs = (h0_ref, h1_ref, h2_ref, h3_ref, h4_ref, h5_ref, h6_ref,
               h7_ref, h8_ref)
    r1 = jax.nn.relu(mid1_ref[...])
    dd = dd_ref[...]
    hs = [None, None, None]
    for g in range(3):
        agg = (agg_ref[g, 0] + agg_ref[g, 1]) * dd[:, g:g + 1]
        pre = EPS * r1 + agg
        hg = jax.nn.relu(jnp.dot(pre, hw1_ref[g], preferred_element_type=jnp.float32))
        hs[g] = hg
        raw2_ref[:, g * NHID:(g + 1) * NHID] = hg
    h48 = jnp.concatenate(hs, axis=1)
    ps_ref[...] = jnp.dot(h48, g2s_ref[...], preferred_element_type=jnp.float32) + gb2_ref[...]
    pd_ref[...] = jnp.dot(h48, g2d_ref[...], preferred_element_type=jnp.float32)
    for g in range(3):
        for c in range(3):
            hs_refs[3 * g + c][...] = hs[c] * dd[:, g:g + 1]


def _tc_mid(mid1, agg1, dd, hw1, g2s, g2d, gb2):
    f32 = jnp.float32
    return pl.pallas_call(
        _tc_mid_body,
        grid=(NBLK,),
        in_specs=[
            pl.BlockSpec((R, NHID), lambda i: (i, 0)),
            pl.BlockSpec((3, 2, R, NHID), lambda i: (0, 0, i, 0)),
            pl.BlockSpec((R, 3), lambda i: (i, 0)),
            pl.BlockSpec((3, NHID, NHID), lambda i: (0, 0, 0)),
            pl.BlockSpec((3 * NHID, 3), lambda i: (0, 0)),
            pl.BlockSpec((3 * NHID, 3), lambda i: (0, 0)),
            pl.BlockSpec((1, 3), lambda i: (0, 0)),
        ],
        out_specs=[pl.BlockSpec((R, 3 * NHID), lambda i: (i, 0))]
        + [pl.BlockSpec((R, NHID), lambda i: (i, 0))] * 9
        + [pl.BlockSpec((R, 3), lambda i: (i, 0))] * 2,
        out_shape=[jax.ShapeDtypeStruct((N, 3 * NHID), f32)]
        + [jax.ShapeDtypeStruct((N, NHID), f32)] * 9
        + [jax.ShapeDtypeStruct((N, 3), f32)] * 2,
    )(mid1, agg1, dd, hw1, g2s, g2d, gb2)


def _tc_post_body(x_ref, mid1_ref, raw2_ref, dd_ref, agg_ref, hw2_ref,
                  t2w_ref, t2b_ref, t3w_ref, t3b_ref,
                  scores_ref, mid2_ref):
    x = x_ref[...]
    x = jnp.where(jnp.isnan(x), jnp.zeros_like(x), x)
    r1 = jax.nn.relu(mid1_ref[...])
    raw2 = raw2_ref[...]
    dd = dd_ref[...]
    t2w = t2w_ref[...]
    acc = jnp.zeros((x.shape[0], NHID), jnp.float32)
    for g in range(3):
        parts = []
        for c in range(3):
            p = 3 * g + c
            parts.append(agg_ref[p, 0] + agg_ref[p, 1])
        agg48 = jnp.concatenate(parts, axis=1) * dd[:, g:g + 1]
        h2g = jax.nn.relu(jnp.dot(EPS * raw2 + agg48, hw2_ref[g],
                                  preferred_element_type=jnp.float32))
        acc = acc + jnp.dot(h2g, t2w[:, g * NHID:(g + 1) * NHID].T,
                            preferred_element_type=jnp.float32)
    acc = acc + jnp.dot(r1, t2w[:, 48:64].T, preferred_element_type=jnp.float32)
    acc = acc + jnp.dot(raw2, t2w[:, 64:112].T, preferred_element_type=jnp.float32)
    acc = acc + jnp.dot(x, t2w[:, 112:240].T, preferred_element_type=jnp.float32)
    mid2 = acc + t2b_ref[...]
    mid2_ref[...] = mid2
    scores_ref[...] = jnp.dot(jax.nn.relu(mid2), t3w_ref[...].T,
                              preferred_element_type=jnp.float32) + t3b_ref[...]


def _tc_post(x, mid1, raw2, dd, agg2, hw2, t2w, t2b, t3w, t3b):
    f32 = jnp.float32
    return pl.pallas_call(
        _tc_post_body,
        grid=(NBLK,),
        in_specs=[
            pl.BlockSpec((R, NFEAT), lambda i: (i, 0)),
            pl.BlockSpec((R, NHID), lambda i: (i, 0)),
            pl.BlockSpec((R, 3 * NHID), lambda i: (i, 0)),
            pl.BlockSpec((R, 3), lambda i: (i, 0)),
            pl.BlockSpec((9, 2, R, NHID), lambda i: (0, 0, i, 0)),
            pl.BlockSpec((3, 3 * NHID, NHID), lambda i: (0, 0, 0)),
            pl.BlockSpec((NHID, 240), lambda i: (0, 0)),
            pl.BlockSpec((1, NHID), lambda i: (0, 0)),
            pl.BlockSpec((NCLASS, NHID), lambda i: (0, 0)),
            pl.BlockSpec((1, NCLASS), lambda i: (0, 0)),
        ],
        out_specs=[
            pl.BlockSpec((R, NCLASS), lambda i: (i, 0)),
            pl.BlockSpec((R, NHID), lambda i: (i, 0)),
        ],
        out_shape=[
            jax.ShapeDtypeStruct((N, NCLASS), f32),
            jax.ShapeDtypeStruct((N, NHID), f32),
        ],
    )(x, mid1, raw2, dd, agg2, hw2, t2w, t2b, t3w, t3b)


def _sc_body(P, gmap, cache, *refs):
    srcg = refs[0:3]
    dstg = refs[3:6]
    pst = refs[6]
    pdt = refs[7]
    hsp = refs[8:8 + P]
    out = refs[8 + P]
    cached = any(c != 0 for c in cache)
    ec = refs[9 + P] if cached else None
    (src_i, dst_i, rows_v, ps_v, pd_v, gate_v, ztmp, acc,
     gsem, isem, ssem, esem) = refs[(10 if cached else 9) + P:]

    cid = lax.axis_index("c")
    sid = lax.axis_index("s")
    wid = cid * NSUB + sid
    nb = jnp.where(wid < EXTRA, NBMIN + 1, NBMIN)
    row0 = wid * NBMIN + jnp.minimum(wid, EXTRA)

    def _zloop(i, _):
        ztmp[i, :] = jnp.zeros((NHID,), jnp.float32)
        return 0
    lax.fori_loop(0, ZR, _zloop, 0)

    for p in range(P):
        g = gmap[p]
        mode = cache[p]
        hs = hsp[p]
        sg = srcg[g]
        dg = dstg[g]

        def _acczero(z, _):
            pltpu.sync_copy(ztmp, acc.at[pl.ds(sid * RPT + z * ZR, ZR)])
            return 0
        lax.fori_loop(0, RPT // ZR, _acczero, 0)
        plsc.subcore_barrier()

        def fire_idx(j, bi):
            base = (row0 + j) * SB
            pltpu.async_copy(sg.at[pl.ds(base, SB)], src_i.at[bi], isem.at[bi])
            pltpu.async_copy(dg.at[pl.ds(base, SB)], dst_i.at[bi], isem.at[bi])

        def wait_idx(j, bi):
            base = (row0 + j) * SB
            pltpu.make_async_copy(sg.at[pl.ds(base, SB)], src_i.at[bi], isem.at[bi]).wait()
            pltpu.make_async_copy(dg.at[pl.ds(base, SB)], dst_i.at[bi], isem.at[bi]).wait()

        def fire(j, b):
            pltpu.async_copy(hs.at[src_i.at[b]], rows_v.at[b], gsem.at[b])
            if mode == 2:
                base = (row0 + j) * SB
                pltpu.async_copy(ec.at[g, pl.ds(base, SB)], gate_v.at[b], gsem.at[b])
            else:
                if mode == 1:
                    @pl.when(j >= 4)
                    def _():
                        bprev = (row0 + j - 4) * SB
                        pltpu.make_async_copy(
                            gate_v.at[b], ec.at[g, pl.ds(bprev, SB)], esem.at[b]).wait()
                pltpu.async_copy(pst.at[src_i.at[b]], ps_v.at[b], gsem.at[b])
                pltpu.async_copy(pdt.at[dst_i.at[b]], pd_v.at[b], gsem.at[b])

        def wait_gathers(j, b):
            pltpu.make_async_copy(hs.at[src_i.at[b]], rows_v.at[b], gsem.at[b]).wait()
            if mode == 2:
                base = (row0 + j) * SB
                pltpu.make_async_copy(ec.at[g, pl.ds(base, SB)], gate_v.at[b], gsem.at[b]).wait()
            else:
                pltpu.make_async_copy(pst.at[src_i.at[b]], ps_v.at[b], gsem.at[b]).wait()
                pltpu.make_async_copy(pdt.at[dst_i.at[b]], pd_v.at[b], gsem.at[b]).wait()

        def fire_scat(b):
            pltpu.async_copy(rows_v.at[b], acc.at[dst_i.at[b]], ssem.at[b], add=True)

        def wait_scat(b):
            pltpu.make_async_copy(rows_v.at[b], acc.at[dst_i.at[b]], ssem.at[b]).wait()

        def process(j, b):
            wait_gathers(j, b)
            lanes = lax.iota(jnp.int32, 16)
            gcol = jnp.full((16,), g, jnp.int32)

            def _rloop(v, _):
                sl = pl.ds(v * 16, 16)
                if mode == 2:
                    ev = gate_v[b, sl]
                else:
                    rows16 = v * 16 + lanes
                    psx = plsc.load_gather(ps_v.at[b], [rows16, gcol])
                    pdx = plsc.load_gather(pd_v.at[b], [rows16, gcol])
                    z = pdx + psx
                    ev = 1.0 - 2.0 / (jnp.exp(z + z) + 1.0)
                    if mode == 1:
                        gate_v[b, sl] = ev
                base = v * 16
                for k in range(16):
                    rows_v[b, base + k, :] = rows_v[b, base + k, :] * ev[k]
                return 0
            lax.fori_loop(0, SB // 16, _rloop, 0)
            fire_scat(b)
            if mode == 1:
                base = (row0 + j) * SB
                pltpu.async_copy(gate_v.at[b], ec.at[g, pl.ds(base, SB)], esem.at[b])

        fire_idx(0, 0)
        fire_idx(1, 1)
        wait_idx(0, 0)
        fire(0, 0)

        def outer(k, _):
            for u in range(4):
                j = 4 * k + u

                @pl.when((j >= 2) & (j + 2 < nb))
                def _():
                    wait_scat((u + 2) % 4)

                @pl.when(j + 2 < nb)
                def _():
                    fire_idx(j + 2, (u + 2) % 4)

                @pl.when(j + 1 < nb)
                def _():
                    wait_idx(j + 1, (u + 1) % 4)
                    fire(j + 1, (u + 1) % 4)

                @pl.when(j < nb)
                def _():
                    process(j, u)
            return 0
        lax.fori_loop(0, TRIPS, outer, 0)

        for s in range(4):
            wait_scat(s)
        if mode == 1:
            for s in range(4):
                pltpu.make_async_copy(
                    gate_v.at[s], ec.at[g, pl.ds(row0 * SB, SB)], esem.at[s]).wait()

        plsc.subcore_barrier()
        pltpu.sync_copy(acc.at[pl.ds(sid * RPT, RPT)],
                        out.at[p, cid, pl.ds(sid * RPT, RPT)])
        plsc.subcore_barrier()


def _sc_passes(P, gmap, cache, srcs, dsts, ps_tab, pd_tab, hs_list):
    cached = any(c != 0 for c in cache)
    out_type = [jax.ShapeDtypeStruct((P, 2, N_PAD, NHID), jnp.float32)]
    if cached:
        out_type.append(jax.ShapeDtypeStruct((3, E), jnp.float32))
    body = functools.partial(_sc_body, P, gmap, cache)
    kern = pl.kernel(
        body,
        out_type=out_type,
        mesh=plsc.VectorSubcoreMesh(core_axis_name="c", subcore_axis_name="s",
                                    num_cores=NCORE, num_subcores=NSUB),
        scratch_types=[
            pltpu.VMEM((4, SB), jnp.int32),
            pltpu.VMEM((4, SB), jnp.int32),
            pltpu.VMEM((4, SB, NHID), jnp.float32),
            pltpu.VMEM((4, SB, 3), jnp.float32),
            pltpu.VMEM((4, SB, 3), jnp.float32),
            pltpu.VMEM((4, SB), jnp.float32),
            pltpu.VMEM((ZR, NHID), jnp.float32),
            pltpu.VMEM_SHARED((N_PAD, NHID), jnp.float32),
            pltpu.SemaphoreType.DMA((4,)),
            pltpu.SemaphoreType.DMA((4,)),
            pltpu.SemaphoreType.DMA((4,)),
            pltpu.SemaphoreType.DMA((4,)),
        ],
        compiler_params=pltpu.CompilerParams(use_tc_tiling_on_sc=False,
                                             needs_layout_passes=False),
    )
    res = kern(*srcs, *dsts, ps_tab, pd_tab, *hs_list)
    return res[0]


def kernel(x, src1, dst1, d1, src2, dst2, d2, src3, dst3, d3,
           t1_W, t1_b, g1_1W, g1_1b, g1_2W, g1_2b, g1_3W, g1_3b,
           hw1_1, hw1_2, hw1_3,
           g2_1W, g2_1b, g2_2W, g2_2b, g2_3W, g2_3b,
           hw2_1, hw2_2, hw2_3, t2_W, t2_b, t3_W, t3_b):
    f32 = jnp.float32
    x = x.astype(f32)
    dd = jnp.stack([d1, d2, d3], axis=1).astype(f32)
    srcs = [src1.astype(jnp.int32), src2.astype(jnp.int32), src3.astype(jnp.int32)]
    dsts = [dst1.astype(jnp.int32), dst2.astype(jnp.int32), dst3.astype(jnp.int32)]

    g1s = jnp.stack([g1_1W[0, NHID:], g1_2W[0, NHID:], g1_3W[0, NHID:]], axis=1)
    g1d = jnp.stack([g1_1W[0, :NHID], g1_2W[0, :NHID], g1_3W[0, :NHID]], axis=1)
    gb1 = jnp.stack([g1_1b[0], g1_2b[0], g1_3b[0]]).reshape(1, 3)
    g2s = jnp.stack([g2_1W[0, 48:], g2_2W[0, 48:], g2_3W[0, 48:]], axis=1)
    g2d = jnp.stack([g2_1W[0, :48], g2_2W[0, :48], g2_3W[0, :48]], axis=1)
    gb2 = jnp.stack([g2_1b[0], g2_2b[0], g2_3b[0]]).reshape(1, 3)

    mid1, hsa, hsb, hsc, ps1, pd1 = _tc_pre(
        x, dd, t1_W, t1_b.reshape(1, NHID), g1s, g1d, gb1)

    agg1 = _sc_passes(
        3, (0, 1, 2), (0, 0, 0), srcs, dsts, ps1, pd1, [hsa, hsb, hsc])

    hw1 = jnp.stack([hw1_1, hw1_2, hw1_3])
    outs = _tc_mid(mid1, agg1, dd, hw1, g2s, g2d, gb2)
    raw2 = outs[0]
    hs2 = outs[1:10]
    ps2, pd2 = outs[10], outs[11]

    agg2 = _sc_passes(
        9, (0, 0, 0, 1, 1, 1, 2, 2, 2), (1, 2, 2, 1, 2, 2, 1, 2, 2),
        srcs, dsts, ps2, pd2, list(hs2))

    hw2 = jnp.stack([hw2_1, hw2_2, hw2_3])
    scores, mid2 = _tc_post(x, mid1, raw2, dd, agg2, hw2,
                            t2_W, t2_b.reshape(1, NHID), t3_W,
                            t3_b.reshape(1, NCLASS))
    return (scores, mid1, mid2)
